# Initial kernel scaffold; baseline (speedup 1.0000x reference)
import jax
import jax.numpy as jnp
from jax.experimental import pallas as pl


def kernel(x, W1, b1, W2, b2, batch):
    raise NotImplementedError("write your pallas kernel here")


# speedup vs baseline: 2.7231x; 2.7231x over previous
import jax
import jax.numpy as jnp
from jax.experimental import pallas as pl
from jax.experimental.pallas import tpu as pltpu

_N_NODES = 100000
_N_GRAPHS = 256
_BLK = 2000
_GRID = _N_NODES // _BLK


def _fused_body(x_ref, ids_ref, W1_ref, b1_ref, W2_ref, b2_ref, out_ref,
                acc_ref):
    i = pl.program_id(0)

    @pl.when(i == 0)
    def _init():
        acc_ref[...] = jnp.zeros_like(acc_ref)

    x = x_ref[...]
    h = jnp.dot(x, W1_ref[...], preferred_element_type=jnp.float32)
    h = jnp.maximum(h + b1_ref[...], 0.0)
    y = jnp.dot(h, W2_ref[...], preferred_element_type=jnp.float32)

    ids = ids_ref[0]
    onehot = (jax.lax.broadcasted_iota(jnp.int32, (_N_GRAPHS, _BLK), 0)
              == ids).astype(jnp.float32)
    yo = jnp.concatenate([y, jnp.ones_like(y)], axis=1)
    acc_ref[...] += jnp.dot(onehot, yo,
                            preferred_element_type=jnp.float32)

    @pl.when(i == _GRID - 1)
    def _finish():
        s = acc_ref[:, 0:1]
        c = acc_ref[:, 1:2]
        out_ref[...] = (s + c * b2_ref[0, 0]) / jnp.maximum(c, 1.0)


def kernel(x, W1, b1, W2, b2, batch):
    ids = batch.astype(jnp.int32).reshape(_GRID, 1, _BLK)
    b1r = b1.reshape(1, -1)
    b2r = b2.reshape(1, 1)
    out = pl.pallas_call(
        _fused_body,
        grid=(_GRID,),
        in_specs=[
            pl.BlockSpec((_BLK, x.shape[1]), lambda i: (i, 0)),
            pl.BlockSpec((1, 1, _BLK), lambda i: (i, 0, 0)),
            pl.BlockSpec(W1.shape, lambda i: (0, 0)),
            pl.BlockSpec((1, b1.shape[0]), lambda i: (0, 0)),
            pl.BlockSpec(W2.shape, lambda i: (0, 0)),
            pl.BlockSpec((1, 1), lambda i: (0, 0)),
        ],
        out_specs=pl.BlockSpec((_N_GRAPHS, 1), lambda i: (0, 0)),
        out_shape=jax.ShapeDtypeStruct((_N_GRAPHS, 1), jnp.float32),
        scratch_shapes=[pltpu.VMEM((_N_GRAPHS, 2), jnp.float32)],
        compiler_params=pltpu.CompilerParams(
            dimension_semantics=("arbitrary",)),
    )(x, ids, W1, b1r, W2, b2r)
    return out


# speedup vs baseline: 2.8926x; 1.0622x over previous
import jax
import jax.numpy as jnp
from jax.experimental import pallas as pl
from jax.experimental.pallas import tpu as pltpu

_N_NODES = 100000
_N_GRAPHS = 256
_BLK = 4000
_GRID = _N_NODES // _BLK


def _fused_body(x_ref, ids_ref, W1_ref, b1_ref, W2_ref, b2_ref, out_ref,
                acc_ref):
    i = pl.program_id(0)

    @pl.when(i == 0)
    def _init():
        acc_ref[...] = jnp.zeros_like(acc_ref)

    x = x_ref[...].astype(jnp.bfloat16)
    h = jnp.dot(x, W1_ref[...].astype(jnp.bfloat16),
                preferred_element_type=jnp.float32)
    h = jnp.maximum(h + b1_ref[...], 0.0).astype(jnp.bfloat16)
    y = jnp.dot(h, W2_ref[...].astype(jnp.bfloat16),
                preferred_element_type=jnp.float32)

    ids = ids_ref[0]
    onehot = (jax.lax.broadcasted_iota(jnp.int32, (_N_GRAPHS, _BLK), 0)
              == ids).astype(jnp.bfloat16)
    yo = jnp.concatenate([y, jnp.ones_like(y)],
                         axis=1).astype(jnp.bfloat16)
    acc_ref[...] += jnp.dot(onehot, yo,
                            preferred_element_type=jnp.float32)

    @pl.when(i == _GRID - 1)
    def _finish():
        s = acc_ref[:, 0:1]
        c = acc_ref[:, 1:2]
        out_ref[...] = (s + c * b2_ref[0, 0]) / jnp.maximum(c, 1.0)


def kernel(x, W1, b1, W2, b2, batch):
    ids = batch.astype(jnp.int32).reshape(_GRID, 1, _BLK)
    b1r = b1.reshape(1, -1)
    b2r = b2.reshape(1, 1)
    out = pl.pallas_call(
        _fused_body,
        grid=(_GRID,),
        in_specs=[
            pl.BlockSpec((_BLK, x.shape[1]), lambda i: (i, 0)),
            pl.BlockSpec((1, 1, _BLK), lambda i: (i, 0, 0)),
            pl.BlockSpec(W1.shape, lambda i: (0, 0)),
            pl.BlockSpec((1, b1.shape[0]), lambda i: (0, 0)),
            pl.BlockSpec(W2.shape, lambda i: (0, 0)),
            pl.BlockSpec((1, 1), lambda i: (0, 0)),
        ],
        out_specs=pl.BlockSpec((_N_GRAPHS, 1), lambda i: (0, 0)),
        out_shape=jax.ShapeDtypeStruct((_N_GRAPHS, 1), jnp.float32),
        scratch_shapes=[pltpu.VMEM((_N_GRAPHS, 2), jnp.float32)],
        compiler_params=pltpu.CompilerParams(
            dimension_semantics=("arbitrary",)),
    )(x, ids, W1, b1r, W2, b2r)
    return out


# speedup vs baseline: 3.0245x; 1.0456x over previous
import jax
import jax.numpy as jnp
from jax.experimental import pallas as pl
from jax.experimental.pallas import tpu as pltpu

_N_NODES = 100000
_N_GRAPHS = 256
_BLK = 10000
_GRID = _N_NODES // _BLK


def _fused_body(x_ref, ids_ref, W1_ref, b1_ref, W2_ref, b2_ref, out_ref,
                acc_ref):
    i = pl.program_id(0)

    @pl.when(i == 0)
    def _init():
        acc_ref[...] = jnp.zeros_like(acc_ref)

    x = x_ref[...].astype(jnp.bfloat16)
    h = jnp.dot(x, W1_ref[...].astype(jnp.bfloat16),
                preferred_element_type=jnp.float32)
    h = jnp.maximum(h + b1_ref[...], 0.0).astype(jnp.bfloat16)
    y = jnp.dot(h, W2_ref[...].astype(jnp.bfloat16),
                preferred_element_type=jnp.float32)

    ids = ids_ref[0]
    onehot = (jax.lax.broadcasted_iota(jnp.int32, (_N_GRAPHS, _BLK), 0)
              == ids).astype(jnp.bfloat16)
    yo = jnp.concatenate([y, jnp.ones_like(y)],
                         axis=1).astype(jnp.bfloat16)
    acc_ref[...] += jnp.dot(onehot, yo,
                            preferred_element_type=jnp.float32)

    @pl.when(i == _GRID - 1)
    def _finish():
        s = acc_ref[:, 0:1]
        c = acc_ref[:, 1:2]
        out_ref[...] = (s + c * b2_ref[0, 0]) / jnp.maximum(c, 1.0)


def kernel(x, W1, b1, W2, b2, batch):
    ids = batch.astype(jnp.int32).reshape(_GRID, 1, _BLK)
    b1r = b1.reshape(1, -1)
    b2r = b2.reshape(1, 1)
    out = pl.pallas_call(
        _fused_body,
        grid=(_GRID,),
        in_specs=[
            pl.BlockSpec((_BLK, x.shape[1]), lambda i: (i, 0)),
            pl.BlockSpec((1, 1, _BLK), lambda i: (i, 0, 0)),
            pl.BlockSpec(W1.shape, lambda i: (0, 0)),
            pl.BlockSpec((1, b1.shape[0]), lambda i: (0, 0)),
            pl.BlockSpec(W2.shape, lambda i: (0, 0)),
            pl.BlockSpec((1, 1), lambda i: (0, 0)),
        ],
        out_specs=pl.BlockSpec((_N_GRAPHS, 1), lambda i: (0, 0)),
        out_shape=jax.ShapeDtypeStruct((_N_GRAPHS, 1), jnp.float32),
        scratch_shapes=[pltpu.VMEM((_N_GRAPHS, 2), jnp.float32)],
        compiler_params=pltpu.CompilerParams(
            dimension_semantics=("arbitrary",)),
    )(x, ids, W1, b1r, W2, b2r)
    return out
